# baseline (speedup 1.0000x reference)
import functools

import jax
import jax.numpy as jnp
from jax import lax
from jax.experimental import pallas as pl
from jax.experimental.pallas import tpu as pltpu
from jax.experimental.pallas import tpu_sc as plsc

_N = 10000
_E = 320000
_D = 128
_G = 64

_NC = 2
_NS = 16
_TILES = _NC * _NS

_CHUNK = 128
_NBUF = 1
_DG = 1
_DS = 1
_T = 8 * (-(-_E // (_CHUNK * _TILES * 8)))
_E_PAD = _CHUNK * _TILES * _T
_EDGES_PER_TILE = _CHUNK * _T
_EDGES_PER_SC = _EDGES_PER_TILE * _NS
_ACC_ROWS = 10240
_ROWS_PER_TILE = _ACC_ROWS // _NS

_CHUNK_DEG = 128
_T_DEG = _E_PAD // (_CHUNK_DEG * _TILES)
_EDGES_PER_TILE_DEG = _CHUNK_DEG * _T_DEG
_EDGES_PER_SC_DEG = _EDGES_PER_TILE_DEG * _NS

_mesh = plsc.VectorSubcoreMesh(core_axis_name="c", subcore_axis_name="s")


@functools.partial(
    pl.kernel,
    out_type=jax.ShapeDtypeStruct((_TILES, _ACC_ROWS), jnp.float32),
    mesh=_mesh,
    scratch_types=[
        pltpu.VMEM((_T_DEG, _CHUNK_DEG), jnp.int32),
        pltpu.VMEM((_ACC_ROWS,), jnp.float32),
    ],
    compiler_params=pltpu.CompilerParams(needs_layout_passes=False),
)
def _sc_degree(dst_hbm, zeros_hbm, out_hbm, idx_a, acc_v):
    c = lax.axis_index("c")
    s = lax.axis_index("s")
    wid = c * _NS + s
    pltpu.sync_copy(zeros_hbm, acc_v)
    ones = jnp.ones((16,), jnp.float32)

    base_chunk = (c * _NS + s) * _T_DEG
    pltpu.sync_copy(dst_hbm.at[pl.ds(base_chunk, _T_DEG)], idx_a)

    def body(g, carry):
        for j in range(_CHUNK_DEG // 16):
            idx = idx_a[g, pl.ds(j * 16, 16)]
            plsc.addupdate_scatter(acc_v, [idx], ones)
        return carry

    lax.fori_loop(0, _T_DEG, body, 0)
    pltpu.sync_copy(acc_v, out_hbm.at[wid])


@functools.partial(
    pl.kernel,
    out_type=jax.ShapeDtypeStruct((_NC, _ACC_ROWS, _D), jnp.float32),
    mesh=_mesh,
    scratch_types=(
        [pltpu.VMEM((_T, _CHUNK), jnp.int32)]
        + [pltpu.VMEM((_CHUNK,), jnp.int32) for _ in range(2)]
        + [pltpu.VMEM((_CHUNK, _D), jnp.float32) for _ in range(2)]
        + [pltpu.VMEM_SHARED((_ACC_ROWS, _D), jnp.float32)]
        + [pltpu.SemaphoreType.DMA for _ in range(4)]
    ),
)
def _sc_aggregate(hs_hbm, src_hbm, dst_hbm, zeros_hbm, out_hbm, *refs):
    src_a = refs[0]
    dst_v = refs[1:3]
    rows_v = refs[3:5]
    acc_s = refs[5]
    gsem = refs[6:8]
    isem = refs[8:10]

    c_ax = lax.axis_index("c")
    s_ax = lax.axis_index("s")
    row0 = s_ax * _ROWS_PER_TILE
    pltpu.sync_copy(zeros_hbm, acc_s.at[pl.ds(row0, _ROWS_PER_TILE)])

    base_chunk = (c_ax * _NS + s_ax) * _T
    pltpu.sync_copy(src_hbm.at[pl.ds(base_chunk, _T)], src_a)
    plsc.subcore_barrier()

    def start_fetch(g, b):
        pltpu.async_copy(dst_hbm.at[base_chunk + g], dst_v[b], isem[b])
        pltpu.async_copy(hs_hbm.at[src_a.at[g]], rows_v[b], gsem[b])

    def wait_fetch(g, b):
        pltpu.make_async_copy(dst_hbm.at[base_chunk + g], dst_v[b],
                              isem[b]).wait()
        pltpu.make_async_copy(hs_hbm.at[src_a.at[g]], rows_v[b],
                              gsem[b]).wait()

    start_fetch(0, 0)

    def body(k, carry):
        for j in range(2):
            g = k * 2 + j
            gn = jnp.minimum(g + 1, _T - 1)
            start_fetch(gn, (j + 1) % 2)
            wait_fetch(g, j)
            pltpu.sync_copy(rows_v[j], acc_s.at[dst_v[j]], add=True)
        return carry

    lax.fori_loop(0, _T // 2, body, 0)
    wait_fetch(_T - 1, 0)

    plsc.subcore_barrier()
    pltpu.sync_copy(
        acc_s.at[pl.ds(row0, _ROWS_PER_TILE)],
        out_hbm.at[c_ax, pl.ds(row0, _ROWS_PER_TILE)],
    )


def _mm(a, b_t):
    return lax.dot_general(a, b_t, (((1,), (1,)), ((), ())),
                           preferred_element_type=jnp.float32)


def _tc0_body(x_ref, w1_ref, h1_ref):
    h1_ref[...] = _mm(x_ref[...], w1_ref[...])


def _tc1_body(h1_ref, degp_ref, hs1_ref, dinv_ref):
    deg = jnp.sum(degp_ref[:, : _N], axis=0) + 1.0
    dinv = lax.rsqrt(deg)
    hs1_ref[...] = h1_ref[...] * dinv[:, None]
    dinv_ref[...] = dinv


def _tc2_body(msgp_ref, hs1_ref, dinv_ref, b1_ref, g1_ref, be1_ref, w2_ref,
              hs2_ref):
    dinv = dinv_ref[...]
    msg = msgp_ref[0, : _N, :] + msgp_ref[1, : _N, :]
    t = dinv[:, None] * (msg + hs1_ref[...]) + b1_ref[...][None, :]
    mu = jnp.mean(t, axis=0)
    var = jnp.mean((t - mu[None, :]) ** 2, axis=0)
    y = (t - mu[None, :]) * lax.rsqrt(var + 1e-5)[None, :] * g1_ref[...][None, :]
    y = jnp.maximum(y + be1_ref[...][None, :], 0.0)
    h2 = _mm(y, w2_ref[...])
    hs2_ref[...] = h2 * dinv[:, None]


def _tc3_body(msgp_ref, hs2_ref, dinv_ref, b2_ref, g2_ref, be2_ref,
              batch_ref, wo_ref, bo_ref, out_ref):
    dinv = dinv_ref[...]
    msg = msgp_ref[0, : _N, :] + msgp_ref[1, : _N, :]
    t = dinv[:, None] * (msg + hs2_ref[...]) + b2_ref[...][None, :]
    mu = jnp.mean(t, axis=0)
    var = jnp.mean((t - mu[None, :]) ** 2, axis=0)
    y = (t - mu[None, :]) * lax.rsqrt(var + 1e-5)[None, :] * g2_ref[...][None, :]
    y = jnp.maximum(y + be2_ref[...][None, :], 0.0)

    gids = lax.broadcasted_iota(jnp.int32, (_N, _G), 1)
    seg = (batch_ref[...][:, None] == gids).astype(jnp.float32)
    sums = lax.dot_general(seg, y, (((0,), (0,)), ((), ())),
                           preferred_element_type=jnp.float32)
    cnt = jnp.sum(seg, axis=0)
    mean = sums / jnp.maximum(cnt, 1.0)[:, None]
    out_ref[...] = _mm(mean, wo_ref[...]) + bo_ref[...][None, :]


def kernel(x, edge_index, batch, W1, b1, g1, be1, W2, b2, g2, be2, Wo, bo):
    pad = _E_PAD - _E
    src = edge_index[0].astype(jnp.int32)
    dst = edge_index[1].astype(jnp.int32)
    ar = jnp.arange(pad, dtype=jnp.int32)
    srcp = jnp.concatenate([src, ar % _N]).reshape(-1, _CHUNK)
    dstp = jnp.concatenate([dst, _N + ar % (_ACC_ROWS - _N)]).reshape(-1, _CHUNK)

    zeros1d = jnp.zeros((_ACC_ROWS,), jnp.float32)
    zerosD = jnp.zeros((_ROWS_PER_TILE, _D), jnp.float32)

    degp = _sc_degree(dstp, zeros1d)

    h1 = pl.pallas_call(
        _tc0_body,
        out_shape=jax.ShapeDtypeStruct((_N, _D), jnp.float32),
    )(x, W1)

    hs1, dinv = pl.pallas_call(
        _tc1_body,
        out_shape=(
            jax.ShapeDtypeStruct((_N, _D), jnp.float32),
            jax.ShapeDtypeStruct((_N,), jnp.float32),
        ),
    )(h1, degp)

    msg1 = _sc_aggregate(hs1, srcp, dstp, zerosD)

    hs2 = pl.pallas_call(
        _tc2_body,
        out_shape=jax.ShapeDtypeStruct((_N, _D), jnp.float32),
    )(msg1, hs1, dinv, b1, g1, be1, W2)

    msg2 = _sc_aggregate(hs2, srcp, dstp, zerosD)

    out = pl.pallas_call(
        _tc3_body,
        out_shape=jax.ShapeDtypeStruct((_G, _D), jnp.float32),
    )(msg2, hs2, dinv, b2, g2, be2, batch.astype(jnp.int32), Wo, bo)
    return out


# speedup vs baseline: 1.0220x; 1.0220x over previous
import functools

import jax
import jax.numpy as jnp
from jax import lax
from jax.experimental import pallas as pl
from jax.experimental.pallas import tpu as pltpu
from jax.experimental.pallas import tpu_sc as plsc

_N = 10000
_E = 320000
_D = 128
_G = 64

_NC = 2
_NS = 16
_TILES = _NC * _NS

_CHUNK = 128
_NBUF = 1
_DG = 1
_DS = 1
_T = 8 * (-(-_E // (_CHUNK * _TILES * 8)))
_E_PAD = _CHUNK * _TILES * _T
_EDGES_PER_TILE = _CHUNK * _T
_EDGES_PER_SC = _EDGES_PER_TILE * _NS
_ACC_ROWS = 10240
_ROWS_PER_TILE = _ACC_ROWS // _NS

_CHUNK_DEG = 128
_T_DEG = _E_PAD // (_CHUNK_DEG * _TILES)
_EDGES_PER_TILE_DEG = _CHUNK_DEG * _T_DEG
_EDGES_PER_SC_DEG = _EDGES_PER_TILE_DEG * _NS

_mesh = plsc.VectorSubcoreMesh(core_axis_name="c", subcore_axis_name="s")


@functools.partial(
    pl.kernel,
    out_type=jax.ShapeDtypeStruct((_TILES, _ACC_ROWS), jnp.float32),
    mesh=_mesh,
    scratch_types=[
        pltpu.VMEM((_T_DEG, _CHUNK_DEG), jnp.int32),
        pltpu.VMEM((_ACC_ROWS,), jnp.float32),
    ],
    compiler_params=pltpu.CompilerParams(needs_layout_passes=False),
)
def _sc_degree(dst_hbm, zeros_hbm, out_hbm, idx_a, acc_v):
    c = lax.axis_index("c")
    s = lax.axis_index("s")
    wid = c * _NS + s
    pltpu.sync_copy(zeros_hbm, acc_v)
    ones = jnp.ones((16,), jnp.float32)

    base_chunk = (c * _NS + s) * _T_DEG
    pltpu.sync_copy(dst_hbm.at[pl.ds(base_chunk, _T_DEG)], idx_a)

    def body(g, carry):
        for j in range(_CHUNK_DEG // 16):
            idx = idx_a[g, pl.ds(j * 16, 16)]
            plsc.addupdate_scatter(acc_v, [idx], ones)
        return carry

    lax.fori_loop(0, _T_DEG, body, 0)
    pltpu.sync_copy(acc_v, out_hbm.at[wid])


@functools.partial(
    pl.kernel,
    out_type=jax.ShapeDtypeStruct((_NC, _ACC_ROWS, _D), jnp.float32),
    mesh=_mesh,
    scratch_types=(
        [pltpu.VMEM((_T, _CHUNK), jnp.int32)]
        + [pltpu.VMEM((_CHUNK,), jnp.int32) for _ in range(2)]
        + [pltpu.VMEM((_CHUNK, _D), jnp.float32) for _ in range(2)]
        + [pltpu.VMEM_SHARED((_ACC_ROWS, _D), jnp.float32)]
        + [pltpu.SemaphoreType.DMA for _ in range(5)]
    ),
)
def _sc_aggregate(hs_hbm, src_hbm, dst_hbm, zeros_hbm, out_hbm, *refs):
    src_a = refs[0]
    dst_v = refs[1:3]
    rows_v = refs[3:5]
    acc_s = refs[5]
    gsem = refs[6:8]
    isem = refs[8:10]
    zsem = refs[10]

    c_ax = lax.axis_index("c")
    s_ax = lax.axis_index("s")
    row0 = s_ax * _ROWS_PER_TILE
    zero_dst = acc_s.at[pl.ds(row0, _ROWS_PER_TILE)]
    pltpu.async_copy(zeros_hbm, zero_dst, zsem)

    base_chunk = (c_ax * _NS + s_ax) * _T
    pltpu.sync_copy(src_hbm.at[pl.ds(base_chunk, _T)], src_a)

    def start_fetch(g, b):
        pltpu.async_copy(dst_hbm.at[base_chunk + g], dst_v[b], isem[b])
        pltpu.async_copy(hs_hbm.at[src_a.at[g]], rows_v[b], gsem[b])

    def wait_fetch(g, b):
        pltpu.make_async_copy(dst_hbm.at[base_chunk + g], dst_v[b],
                              isem[b]).wait()
        pltpu.make_async_copy(hs_hbm.at[src_a.at[g]], rows_v[b],
                              gsem[b]).wait()

    start_fetch(0, 0)
    pltpu.make_async_copy(zeros_hbm, zero_dst, zsem).wait()
    plsc.subcore_barrier()

    def body(k, carry):
        for j in range(2):
            g = k * 2 + j
            gn = jnp.minimum(g + 1, _T - 1)
            start_fetch(gn, (j + 1) % 2)
            wait_fetch(g, j)
            pltpu.sync_copy(rows_v[j], acc_s.at[dst_v[j]], add=True)
        return carry

    lax.fori_loop(0, _T // 2, body, 0)
    wait_fetch(_T - 1, 0)

    plsc.subcore_barrier()
    pltpu.sync_copy(
        acc_s.at[pl.ds(row0, _ROWS_PER_TILE)],
        out_hbm.at[c_ax, pl.ds(row0, _ROWS_PER_TILE)],
    )


def _mm(a, b_t):
    return lax.dot_general(a, b_t, (((1,), (1,)), ((), ())),
                           preferred_element_type=jnp.float32)


def _tc1_body(x_ref, w1_ref, degp_ref, hs1_ref, dinv_ref):
    deg = jnp.sum(degp_ref[:, : _N], axis=0) + 1.0
    dinv = lax.rsqrt(deg)
    h1 = _mm(x_ref[...], w1_ref[...])
    hs1_ref[...] = h1 * dinv[:, None]
    dinv_ref[...] = dinv


def _tc2_body(msgp_ref, hs1_ref, dinv_ref, b1_ref, g1_ref, be1_ref, w2_ref,
              hs2_ref):
    dinv = dinv_ref[...]
    msg = msgp_ref[0, : _N, :] + msgp_ref[1, : _N, :]
    t = dinv[:, None] * (msg + hs1_ref[...]) + b1_ref[...][None, :]
    mu = jnp.mean(t, axis=0)
    var = jnp.mean((t - mu[None, :]) ** 2, axis=0)
    y = (t - mu[None, :]) * lax.rsqrt(var + 1e-5)[None, :] * g1_ref[...][None, :]
    y = jnp.maximum(y + be1_ref[...][None, :], 0.0)
    h2 = _mm(y, w2_ref[...])
    hs2_ref[...] = h2 * dinv[:, None]


def _tc3_body(msgp_ref, hs2_ref, dinv_ref, b2_ref, g2_ref, be2_ref,
              batch_ref, wo_ref, bo_ref, out_ref):
    dinv = dinv_ref[...]
    msg = msgp_ref[0, : _N, :] + msgp_ref[1, : _N, :]
    t = dinv[:, None] * (msg + hs2_ref[...]) + b2_ref[...][None, :]
    mu = jnp.mean(t, axis=0)
    var = jnp.mean((t - mu[None, :]) ** 2, axis=0)
    y = (t - mu[None, :]) * lax.rsqrt(var + 1e-5)[None, :] * g2_ref[...][None, :]
    y = jnp.maximum(y + be2_ref[...][None, :], 0.0)

    gids = lax.broadcasted_iota(jnp.int32, (_N, _G), 1)
    seg = (batch_ref[...][:, None] == gids).astype(jnp.float32)
    sums = lax.dot_general(seg, y, (((0,), (0,)), ((), ())),
                           preferred_element_type=jnp.float32)
    cnt = jnp.sum(seg, axis=0)
    mean = sums / jnp.maximum(cnt, 1.0)[:, None]
    out_ref[...] = _mm(mean, wo_ref[...]) + bo_ref[...][None, :]


def kernel(x, edge_index, batch, W1, b1, g1, be1, W2, b2, g2, be2, Wo, bo):
    pad = _E_PAD - _E
    src = edge_index[0].astype(jnp.int32)
    dst = edge_index[1].astype(jnp.int32)
    ar = jnp.arange(pad, dtype=jnp.int32)
    srcp = jnp.concatenate([src, ar % _N]).reshape(-1, _CHUNK)
    dstp = jnp.concatenate([dst, _N + ar % (_ACC_ROWS - _N)]).reshape(-1, _CHUNK)

    zeros1d = jnp.zeros((_ACC_ROWS,), jnp.float32)
    zerosD = jnp.zeros((_ROWS_PER_TILE, _D), jnp.float32)

    degp = _sc_degree(dstp, zeros1d)

    hs1, dinv = pl.pallas_call(
        _tc1_body,
        out_shape=(
            jax.ShapeDtypeStruct((_N, _D), jnp.float32),
            jax.ShapeDtypeStruct((_N,), jnp.float32),
        ),
    )(x, W1, degp)

    msg1 = _sc_aggregate(hs1, srcp, dstp, zerosD)

    hs2 = pl.pallas_call(
        _tc2_body,
        out_shape=jax.ShapeDtypeStruct((_N, _D), jnp.float32),
    )(msg1, hs1, dinv, b1, g1, be1, W2)

    msg2 = _sc_aggregate(hs2, srcp, dstp, zerosD)

    out = pl.pallas_call(
        _tc3_body,
        out_shape=jax.ShapeDtypeStruct((_G, _D), jnp.float32),
    )(msg2, hs2, dinv, b2, g2, be2, batch.astype(jnp.int32), Wo, bo)
    return out
